# baseline (speedup 1.0000x reference)
import functools

import jax
import jax.numpy as jnp
from jax import lax
from jax.experimental import pallas as pl
from jax.experimental.pallas import tpu as pltpu
from jax.experimental.pallas import tpu_sc as plsc

_GAMMA = 24.0
_L = 16


def kernel(entity_table, relation_table, positive_sample, negative_sample,
           q_entity, q_relation):
    B = positive_sample.shape[0]
    NNEG = negative_sample.shape[1]
    D = entity_table.shape[1]
    D2 = 2 * D
    info = plsc.get_sparse_core_info()
    NW = info.num_cores * info.num_subcores
    BPW = B // NW
    DC = D // _L
    RPC = 2
    NCH = BPW // RPC
    CR = RPC * NNEG

    ent2 = entity_table.reshape(entity_table.shape[0] // 2, D2)
    rel2 = relation_table.reshape(relation_table.shape[0] // 2, D2)

    mesh = plsc.VectorSubcoreMesh(core_axis_name="c", subcore_axis_name="s")

    @functools.partial(
        pl.kernel,
        out_type=(jax.ShapeDtypeStruct((B,), jnp.float32),
                  jax.ShapeDtypeStruct((B, NNEG), jnp.float32)),
        mesh=mesh,
        compiler_params=pltpu.CompilerParams(
            needs_layout_passes=False, use_tc_tiling_on_sc=False),
        scratch_types=[
            pltpu.VMEM((BPW,), jnp.int32),
            pltpu.VMEM((BPW,), jnp.int32),
            pltpu.VMEM((BPW,), jnp.int32),
            pltpu.VMEM((BPW,), jnp.int32),
            pltpu.VMEM((BPW, NNEG), jnp.int32),
            pltpu.VMEM((CR,), jnp.int32),
            pltpu.VMEM((CR,), jnp.int32),
            pltpu.VMEM((BPW, D2), jnp.float32),
            pltpu.VMEM((BPW, D), jnp.float32),
            pltpu.VMEM((CR, D2), jnp.float32),
            pltpu.VMEM((CR, D2), jnp.float32),
            pltpu.VMEM((BPW,), jnp.float32),
            pltpu.VMEM((BPW, NNEG), jnp.float32),
            pltpu.SemaphoreType.DMA,
            pltpu.SemaphoreType.DMA,
            pltpu.SemaphoreType.DMA,
        ],
    )
    def _gqe(ent_hbm, rel_hbm, pos_hbm, neg_hbm, qe_hbm, qr_hbm,
             out_pos_hbm, out_neg_hbm,
             qe_idx, qr_idx, pos_idx, half_idx, neg_idx, cidx0, cidx1,
             pair_buf, q_rows, nbuf0, nbuf1, out_pos, out_neg,
             sem, nsem0, nsem1):
        wid = lax.axis_index("s") * info.num_cores + lax.axis_index("c")
        base = wid * BPW
        iota = lax.iota(jnp.int32, _L)

        pltpu.sync_copy(qe_hbm.at[pl.ds(base, BPW)], qe_idx)
        pltpu.sync_copy(qr_hbm.at[pl.ds(base, BPW)], qr_idx)
        pltpu.sync_copy(pos_hbm.at[pl.ds(base, BPW)], pos_idx)
        pltpu.sync_copy(neg_hbm.at[pl.ds(base, BPW)], neg_idx)

        def halve(src, dst, n):
            @pl.loop(0, n // _L)
            def _(k):
                sl = pl.ds(k * _L, _L)
                dst[sl] = lax.shift_right_logical(src[sl], 1)

        def fire_chunk(c, cidx, buf, nsem):
            for i in range(RPC):
                @pl.loop(0, NNEG // _L)
                def _(k, i=i):
                    sl = pl.ds(i * NNEG + k * _L, _L)
                    cidx[sl] = lax.shift_right_logical(
                        neg_idx[c * RPC + i, pl.ds(k * _L, _L)], 1)
            for i in range(RPC):
                pltpu.async_copy(
                    ent_hbm.at[cidx.at[pl.ds(i * NNEG, NNEG)]],
                    buf.at[pl.ds(i * NNEG, NNEG)], nsem)

        def drain_chunk(buf, nsem):
            pltpu.make_async_copy(ent_hbm.at[pl.ds(0, CR)], buf, nsem).wait()

        def row_l1(buf, r, off, qv):
            acc = jnp.abs(buf[r, pl.ds(off, _L)] - qv[0])
            for cc in range(1, DC):
                acc = acc + jnp.abs(buf[r, pl.ds(off + cc * _L, _L)] - qv[cc])
            return jnp.sum(acc, axis=0)

        def compute_chunk(c, buf):
            @pl.loop(0, RPC)
            def _(i):
                row = c * RPC + i
                qv = [q_rows[row, pl.ds(cc * _L, _L)] for cc in range(DC)]
                for j in range(NNEG):
                    if j % _L == 0:
                        ev = neg_idx[row, pl.ds(j, _L)]
                        res = jnp.zeros((_L,), jnp.float32)
                    r = i * NNEG + j
                    off = (ev[j % _L] & 1) * D
                    s = row_l1(buf, r, off, qv)
                    res = jnp.where(iota == (j % _L), _GAMMA - s, res)
                    if j % _L == _L - 1:
                        out_neg[row, pl.ds((j // _L) * _L, _L)] = res

        fire_chunk(0, cidx0, nbuf0, nsem0)

        halve(qe_idx, half_idx, BPW)
        pltpu.async_copy(ent_hbm.at[half_idx], pair_buf, sem).wait()

        @pl.loop(0, BPW // _L)
        def _(g):
            ev = qe_idx[pl.ds(g * _L, _L)]
            for k in range(_L):
                row = g * _L + k
                off = (ev[k] & 1) * D
                for cc in range(DC):
                    q_rows[row, pl.ds(cc * _L, _L)] = (
                        pair_buf[row, pl.ds(off + cc * _L, _L)])

        halve(qr_idx, half_idx, BPW)
        pltpu.async_copy(rel_hbm.at[half_idx], pair_buf, sem).wait()

        @pl.loop(0, BPW // _L)
        def _(g):
            ev = qr_idx[pl.ds(g * _L, _L)]
            for k in range(_L):
                row = g * _L + k
                off = (ev[k] & 1) * D
                for cc in range(DC):
                    sl = pl.ds(cc * _L, _L)
                    q_rows[row, sl] = (q_rows[row, sl]
                                       + pair_buf[row, pl.ds(off + cc * _L, _L)])

        halve(pos_idx, half_idx, BPW)
        pltpu.async_copy(ent_hbm.at[half_idx], pair_buf, sem).wait()

        @pl.loop(0, BPW // _L)
        def _(g):
            ev = pos_idx[pl.ds(g * _L, _L)]
            res = jnp.zeros((_L,), jnp.float32)
            for k in range(_L):
                row = g * _L + k
                qv = [q_rows[row, pl.ds(cc * _L, _L)] for cc in range(DC)]
                off = (ev[k] & 1) * D
                s = row_l1(pair_buf, row, off, qv)
                res = jnp.where(iota == k, _GAMMA - s, res)
            out_pos[pl.ds(g * _L, _L)] = res

        @pl.loop(0, NCH // 2)
        def _(t):
            c0 = 2 * t
            fire_chunk(c0 + 1, cidx1, nbuf1, nsem1)
            drain_chunk(nbuf0, nsem0)
            compute_chunk(c0, nbuf0)

            @pl.when(c0 + 2 < NCH)
            def _():
                fire_chunk(c0 + 2, cidx0, nbuf0, nsem0)

            drain_chunk(nbuf1, nsem1)
            compute_chunk(c0 + 1, nbuf1)

        pltpu.sync_copy(out_pos, out_pos_hbm.at[pl.ds(base, BPW)])
        pltpu.sync_copy(out_neg, out_neg_hbm.at[pl.ds(base, BPW)])

    return _gqe(ent2, rel2, positive_sample, negative_sample,
                q_entity, q_relation)


# speedup vs baseline: 1.4998x; 1.4998x over previous
import functools

import jax
import jax.numpy as jnp
from jax import lax
from jax.experimental import pallas as pl
from jax.experimental.pallas import tpu as pltpu
from jax.experimental.pallas import tpu_sc as plsc

_GAMMA = 24.0
_L = 16


def kernel(entity_table, relation_table, positive_sample, negative_sample,
           q_entity, q_relation):
    B = positive_sample.shape[0]
    NNEG = negative_sample.shape[1]
    D = entity_table.shape[1]
    D2 = 2 * D
    info = plsc.get_sparse_core_info()
    NW = info.num_cores * info.num_subcores
    BPW = B // NW
    DC = D // _L
    RPC = 1
    NCH = BPW // RPC
    CR = RPC * NNEG

    ent2 = jnp.concatenate([entity_table, entity_table], axis=1)
    rel2 = jnp.concatenate([relation_table, relation_table], axis=1)

    mesh = plsc.VectorSubcoreMesh(core_axis_name="c", subcore_axis_name="s")

    @functools.partial(
        pl.kernel,
        out_type=(jax.ShapeDtypeStruct((B,), jnp.float32),
                  jax.ShapeDtypeStruct((B, NNEG), jnp.float32)),
        mesh=mesh,
        compiler_params=pltpu.CompilerParams(
            needs_layout_passes=False, use_tc_tiling_on_sc=True),
        scratch_types=[
            pltpu.VMEM((BPW,), jnp.int32),
            pltpu.VMEM((BPW,), jnp.int32),
            pltpu.VMEM((BPW,), jnp.int32),
            pltpu.VMEM((BPW, NNEG), jnp.int32),
            pltpu.VMEM((BPW, D2), jnp.float32),
            pltpu.VMEM((BPW, D), jnp.float32),
            pltpu.VMEM((CR, D2), jnp.float32),
            pltpu.VMEM((CR, D2), jnp.float32),
            pltpu.VMEM((BPW,), jnp.float32),
            pltpu.VMEM((BPW, NNEG), jnp.float32),
            pltpu.SemaphoreType.DMA,
            pltpu.SemaphoreType.DMA,
            pltpu.SemaphoreType.DMA,
        ],
    )
    def _gqe(ent_hbm, rel_hbm, pos_hbm, neg_hbm, qe_hbm, qr_hbm,
             out_pos_hbm, out_neg_hbm,
             qe_idx, qr_idx, pos_idx, neg_idx, full_buf, q_rows,
             nbuf0, nbuf1, out_pos, out_neg, sem, nsem0, nsem1):
        wid = lax.axis_index("s") * info.num_cores + lax.axis_index("c")
        base = wid * BPW
        iota = lax.iota(jnp.int32, _L)

        pltpu.sync_copy(qe_hbm.at[pl.ds(base, BPW)], qe_idx)
        pltpu.sync_copy(qr_hbm.at[pl.ds(base, BPW)], qr_idx)
        pltpu.sync_copy(pos_hbm.at[pl.ds(base, BPW)], pos_idx)
        pltpu.sync_copy(neg_hbm.at[pl.ds(base, BPW)], neg_idx)

        def fire_chunk(c, buf, nsem):
            for i in range(RPC):
                pltpu.async_copy(ent_hbm.at[neg_idx.at[c * RPC + i]],
                                 buf.at[pl.ds(i * NNEG, NNEG)], nsem)

        def drain_chunk(buf, nsem):
            pltpu.make_async_copy(ent_hbm.at[pl.ds(0, CR)], buf, nsem).wait()

        def row_l1(buf, r, qv):
            acc = jnp.abs(buf[r, pl.ds(0, _L)] - qv[0])
            for cc in range(1, DC):
                acc = acc + jnp.abs(buf[r, pl.ds(cc * _L, _L)] - qv[cc])
            return jnp.sum(acc, axis=0)

        def compute_chunk(c, buf):
            @pl.loop(0, RPC)
            def _(i):
                row = c * RPC + i
                qv = [q_rows[row, pl.ds(cc * _L, _L)] for cc in range(DC)]
                res = jnp.zeros((_L,), jnp.float32)
                for j in range(NNEG):
                    s = row_l1(buf, i * NNEG + j, qv)
                    res = jnp.where(iota == (j % _L), _GAMMA - s, res)
                    if j % _L == _L - 1:
                        out_neg[row, pl.ds((j // _L) * _L, _L)] = res

        fire_chunk(0, nbuf0, nsem0)

        pltpu.async_copy(ent_hbm.at[qe_idx], full_buf, sem).wait()

        @pl.loop(0, BPW)
        def _(r):
            for cc in range(DC):
                q_rows[r, pl.ds(cc * _L, _L)] = full_buf[r, pl.ds(cc * _L, _L)]

        pltpu.async_copy(rel_hbm.at[qr_idx], full_buf, sem).wait()

        @pl.loop(0, BPW)
        def _(r):
            for cc in range(DC):
                sl = pl.ds(cc * _L, _L)
                q_rows[r, sl] = q_rows[r, sl] + full_buf[r, sl]

        pltpu.async_copy(ent_hbm.at[pos_idx], full_buf, sem).wait()

        @pl.loop(0, BPW // _L)
        def _(g):
            res = jnp.zeros((_L,), jnp.float32)
            for k in range(_L):
                row = g * _L + k
                qv = [q_rows[row, pl.ds(cc * _L, _L)] for cc in range(DC)]
                s = row_l1(full_buf, row, qv)
                res = jnp.where(iota == k, _GAMMA - s, res)
            out_pos[pl.ds(g * _L, _L)] = res

        @pl.loop(0, NCH // 2)
        def _(t):
            c0 = 2 * t
            fire_chunk(c0 + 1, nbuf1, nsem1)
            drain_chunk(nbuf0, nsem0)
            compute_chunk(c0, nbuf0)

            @pl.when(c0 + 2 < NCH)
            def _():
                fire_chunk(c0 + 2, nbuf0, nsem0)

            drain_chunk(nbuf1, nsem1)
            compute_chunk(c0 + 1, nbuf1)

        pltpu.sync_copy(out_pos, out_pos_hbm.at[pl.ds(base, BPW)])
        pltpu.sync_copy(out_neg, out_neg_hbm.at[pl.ds(base, BPW)])

    return _gqe(ent2, rel2, positive_sample, negative_sample,
                q_entity, q_relation)
